# baseline (speedup 1.0000x reference)
import functools

import jax
import jax.numpy as jnp
from jax import lax
from jax.experimental import pallas as pl
from jax.experimental.pallas import tpu as pltpu
from jax.experimental.pallas import tpu_sc as plsc

_INFO = plsc.get_sparse_core_info()
_NC, _NS = _INFO.num_cores, _INFO.num_subcores
_NW = _NC * _NS

_TOK = 4


@functools.cache
def _build(T, batch, v, d):
    ncb = d // 128
    nseg = ncb * batch
    n_active = T // _TOK
    mesh = plsc.VectorSubcoreMesh(core_axis_name="c", subcore_axis_name="s")

    @functools.partial(
        pl.kernel,
        out_type=jax.ShapeDtypeStruct((T * nseg, 128), jnp.float32),
        mesh=mesh,
        scratch_types=[
            pltpu.VMEM((_TOK * batch,), jnp.int32),
            pltpu.VMEM((_TOK * nseg,), jnp.int32),
            pltpu.VMEM((_TOK * nseg, 128), jnp.float32),
            pltpu.SemaphoreType.DMA,
            pltpu.SemaphoreType.DMA,
        ],
    )
    def gather_kernel(iv_hbm, table_hbm, out_hbm, base_v, idx_v, segs_v, sem, sem_w):
        wid = lax.axis_index("s") * _NC + lax.axis_index("c")

        @pl.when(wid < n_active)
        def _():
            t0 = wid * _TOK
            pltpu.sync_copy(iv_hbm.at[pl.ds(t0 * batch, _TOK * batch)], base_v)
            lane = lax.iota(jnp.int32, 16)
            b = lane & (batch - 1)
            bvec = base_v[...]
            for t in range(_TOK):
                patt = jnp.full((16,), bvec[t * batch], jnp.int32)
                for bi in range(1, batch):
                    patt = jnp.where(b == bi, bvec[t * batch + bi], patt)
                for ch in range(nseg // 16):
                    cb = (ch * 16 + lane) >> 2
                    idx_v[pl.ds(t * nseg + ch * 16, 16)] = patt * ncb + cb
            gathers = [
                pltpu.async_copy(
                    table_hbm.at[idx_v.at[pl.ds(k * nseg, nseg)]],
                    segs_v.at[pl.ds(k * nseg, nseg)],
                    sem,
                )
                for k in range(_TOK)
            ]
            writes = []
            for k in range(_TOK):
                gathers[k].wait()
                writes.append(
                    pltpu.async_copy(
                        segs_v.at[pl.ds(k * nseg, nseg)],
                        out_hbm.at[pl.ds((t0 + k) * nseg, nseg)],
                        sem_w,
                    )
                )
            for w in writes:
                w.wait()

    return gather_kernel


def kernel(indices, embedding):
    batch, t = indices.shape
    v, d = embedding.shape
    ncb = d // 128
    iv = indices.astype(jnp.int32).T.reshape(t * batch)
    table2 = embedding.reshape(v * ncb, 128)
    out = _build(t, batch, v, d)(iv, table2)
    return (
        out.reshape(t, ncb, batch, 128).transpose(2, 0, 1, 3).reshape(batch, t, d)
    )
